# baseline (speedup 1.0000x reference)
import jax
import jax.numpy as jnp
from jax.experimental import pallas as pl


def _add_body(x_ref, pe_ref, o_ref):
    o_ref[0] = x_ref[0] + pe_ref[...]


def kernel(x, pe):
    B, S, D = x.shape
    max_seq = pe.shape[1]
    stride = max_seq // S
    pe2 = pe[:, : S * stride, :].reshape(S, stride * D)

    S_BLK = 256
    grid = (S // S_BLK, B)
    return pl.pallas_call(
        _add_body,
        grid=grid,
        in_specs=[
            pl.BlockSpec((1, S_BLK, D), lambda j, b: (b, j, 0)),
            pl.BlockSpec((S_BLK, D), lambda j, b: (j, 0)),
        ],
        out_specs=pl.BlockSpec((1, S_BLK, D), lambda j, b: (b, j, 0)),
        out_shape=jax.ShapeDtypeStruct((B, S, D), x.dtype),
    )(x, pe2)


# speedup vs baseline: 1.1786x; 1.1786x over previous
import jax
import jax.numpy as jnp
from jax.experimental import pallas as pl


def _add_body(x_ref, pe_ref, o_ref):
    o_ref[...] = x_ref[...] + pe_ref[...]


def kernel(x, pe):
    B, S, D = x.shape
    max_seq = pe.shape[1]
    stride = max_seq // S
    pe2 = pe[:, : S * stride, :].reshape(S, stride * D)

    S_BLK = 256
    grid = (S // S_BLK,)
    out = pl.pallas_call(
        _add_body,
        grid=grid,
        in_specs=[
            pl.BlockSpec((B, S_BLK, D), lambda j: (0, j, 0)),
            pl.BlockSpec((S_BLK, D), lambda j: (j, 0)),
        ],
        out_specs=pl.BlockSpec((B, S_BLK, D), lambda j: (0, j, 0)),
        out_shape=jax.ShapeDtypeStruct((B, S, D), x.dtype),
    )(x, pe2)
    return out
